# baseline (speedup 1.0000x reference)
import functools

import jax
import jax.numpy as jnp
import numpy as np
from jax import lax
from jax.experimental import pallas as pl
from jax.experimental.pallas import tpu as pltpu
from jax.experimental.pallas import tpu_sc as plsc

_FIELD_DIMS = [100000] * 26
_B = 16384
_F = 26
_D = 16
_TOTAL = sum(_FIELD_DIMS)
_MLP_IN = _F * _D
_BF = _B * _F

_NC = 2
_NS = 16
_NW = _NC * _NS
_PER_W = _BF // _NW
_CHUNK = 128
_NCHUNK = _PER_W // _CHUNK


def _sc_gather(xi, emb, lin_flat):
    mesh = plsc.VectorSubcoreMesh(core_axis_name="c", subcore_axis_name="s")

    @functools.partial(
        pl.kernel,
        mesh=mesh,
        compiler_params=pltpu.CompilerParams(use_tc_tiling_on_sc=False),
        out_type=[
            jax.ShapeDtypeStruct((_BF, _D), jnp.float32),
            jax.ShapeDtypeStruct((_BF,), jnp.float32),
        ],
        scratch_types=[
            pltpu.VMEM((_NCHUNK, _CHUNK), jnp.int32),
        ] + [pltpu.VMEM((_CHUNK, _D), jnp.float32)] * 4
          + [pltpu.VMEM((_CHUNK,), jnp.float32)] * 4
          + [pltpu.SemaphoreType.DMA] * 8,
    )
    def gather_k(xi_hbm, emb_hbm, lin_hbm, e_out, lin_out, idx_v,
                 r0, r1, r2, r3, l0, l1, l2, l3,
                 se0, se1, se2, se3, sl0, sl1, sl2, sl3):
        wid = lax.axis_index("s") * _NC + lax.axis_index("c")
        base = wid * _PER_W
        pltpu.sync_copy(xi_hbm.at[wid], idx_v)

        rows = [r0, r1, r2, r3]
        lins = [l0, l1, l2, l3]
        sems_e = [se0, se1, se2, se3]
        sems_l = [sl0, sl1, sl2, sl3]

        def fire(k, j):
            pltpu.async_copy(emb_hbm.at[idx_v.at[j]], rows[k], sems_e[k])
            pltpu.async_copy(lin_hbm.at[idx_v.at[j]], lins[k], sems_l[k])

        def drain_write(k, j):
            pltpu.make_async_copy(e_out.at[pl.ds(0, _CHUNK)], rows[k], sems_e[k]).wait()
            pltpu.make_async_copy(lin_out.at[pl.ds(0, _CHUNK)], lins[k], sems_l[k]).wait()
            off = base + j * _CHUNK
            pltpu.sync_copy(rows[k], e_out.at[pl.ds(off, _CHUNK)])
            pltpu.sync_copy(lins[k], lin_out.at[pl.ds(off, _CHUNK)])

        for k in range(3):
            fire(k, k)

        def body(t, carry):
            for i in range(4):
                c = 4 * t + i

                @pl.when(c + 3 < _NCHUNK)
                def _(c=c, i=i):
                    fire((i + 3) % 4, c + 3)

                drain_write(i, c)
            return carry

        lax.fori_loop(0, _NCHUNK // 4, body, 0)

    return gather_k(xi, emb, lin_flat)


_BB = 512


def _tc_body(e_ref, lin_ref, s_ref, w1_ref, b1_ref, w2_ref, b2_ref, w3_ref, cb_ref, out_ref):
    e = e_ref[...]
    s = jnp.dot(e, s_ref[...], preferred_element_type=jnp.float32)
    fm = 0.5 * (jnp.sum(s * s, axis=1) - jnp.sum(e * e, axis=1))
    fm_logit = jax.nn.sigmoid(fm)
    lin = jnp.sum(lin_ref[...], axis=1)
    h = jnp.dot(e, w1_ref[...], preferred_element_type=jnp.float32) + b1_ref[...]
    h = jnp.maximum(h, 0.0)
    h = jnp.dot(h, w2_ref[...], preferred_element_type=jnp.float32) + b2_ref[...]
    h = jnp.maximum(h, 0.0)
    dnn = jnp.dot(h, w3_ref[...], preferred_element_type=jnp.float32)[:, 0]
    logit = lin + fm_logit + dnn + cb_ref[0, 0]
    out_ref[...] = jax.nn.sigmoid(logit)


def _tc_dense(e2, linv, s_mat, W1, b1, W2, b2, W3, cb, *, interpret=False):
    grid = (_B // _BB,)
    full = lambda shape: pl.BlockSpec(shape, lambda i: (0,) * len(shape))
    return pl.pallas_call(
        _tc_body,
        grid=grid,
        in_specs=[
            pl.BlockSpec((_BB, _MLP_IN), lambda i: (i, 0)),
            pl.BlockSpec((_BB, _F), lambda i: (i, 0)),
            full((_MLP_IN, _D)),
            full((_MLP_IN, 256)),
            full((1, 256)),
            full((256, 128)),
            full((1, 128)),
            full((128, 1)),
            full((1, 1)),
        ],
        out_specs=pl.BlockSpec((_BB,), lambda i: (i,)),
        out_shape=jax.ShapeDtypeStruct((_B,), jnp.float32),
        interpret=interpret,
    )(e2, linv, s_mat, W1, b1, W2, b2, W3, cb)


_OFFSETS = np.cumsum([0] + _FIELD_DIMS[:-1]).astype(np.int32)
_S_MAT = np.tile(np.eye(_D, dtype=np.float32), (_F, 1))


def kernel(x, emb, lin_w, lin_b, W1, b1, W2, b2, W3, b3):
    xi = (x + _OFFSETS[None, :]).reshape(_NW, _NCHUNK, _CHUNK)
    e_flat, lin_vals = _sc_gather(xi, emb, lin_w.reshape(-1))
    e2 = e_flat.reshape(_B, _MLP_IN)
    linv = lin_vals.reshape(_B, _F)
    cb = (lin_b + b3).reshape(1, 1)
    return _tc_dense(
        e2, linv, jnp.asarray(_S_MAT), W1, b1.reshape(1, 256), W2, b2.reshape(1, 128), W3, cb
    )
